# baseline (speedup 1.0000x reference)
import jax
import jax.numpy as jnp
from jax.experimental import pallas as pl

_B, _H = 16384, 200
_FLAT_ROWS = _B * _H // 256
_ROW_BLK = 256


def _body(xx_ref, xy_ref, o_ref):
    scale = jnp.float32(2.0 / 3.0)
    fx = xx_ref[...].astype(jnp.float32) * scale - 1.0
    fy = xy_ref[...].astype(jnp.float32) * scale - 1.0
    o_ref[...] = jnp.stack((fx, fy), axis=-1).reshape(fx.shape[0], 512)


def kernel(x_x, x_y):
    xf = x_x.reshape(_FLAT_ROWS, 256)
    yf = x_y.reshape(_FLAT_ROWS, 256)
    grid = (_FLAT_ROWS // _ROW_BLK,)
    spec = pl.BlockSpec((_ROW_BLK, 256), lambda i: (i, 0))
    ospec = pl.BlockSpec((_ROW_BLK, 512), lambda i: (i, 0))
    o = pl.pallas_call(
        _body,
        grid=grid,
        in_specs=[spec, spec],
        out_specs=ospec,
        out_shape=jax.ShapeDtypeStruct((_FLAT_ROWS, 512), jnp.float32),
    )(xf, yf)
    return o.reshape(_B, _H, 2).astype(jnp.complex64)


# speedup vs baseline: 37.5569x; 37.5569x over previous
import jax
import jax.numpy as jnp
from jax.experimental import pallas as pl

_B, _H = 16384, 200
_FLAT_ROWS = _B * _H // 256
_ROW_BLK = 256


def _body(xx_ref, xy_ref, rx_ref, ry_ref):
    scale = jnp.float32(2.0 / 3.0)
    rx_ref[...] = xx_ref[...].astype(jnp.float32) * scale - 1.0
    ry_ref[...] = xy_ref[...].astype(jnp.float32) * scale - 1.0


def kernel(x_x, x_y):
    xf = x_x.reshape(_FLAT_ROWS, 256)
    yf = x_y.reshape(_FLAT_ROWS, 256)
    grid = (_FLAT_ROWS // _ROW_BLK,)
    spec = pl.BlockSpec((_ROW_BLK, 256), lambda i: (i, 0))
    rx, ry = pl.pallas_call(
        _body,
        grid=grid,
        in_specs=[spec, spec],
        out_specs=[spec, spec],
        out_shape=[jax.ShapeDtypeStruct((_FLAT_ROWS, 256), jnp.float32)] * 2,
    )(xf, yf)
    out = jnp.stack((rx.reshape(_B, _H), ry.reshape(_B, _H)), axis=-1)
    return out.astype(jnp.complex64)


# speedup vs baseline: 58.4918x; 1.5574x over previous
import jax
import jax.numpy as jnp
from jax.experimental import pallas as pl

_B, _H = 16384, 200
_HBLK = 8


def _body(xx_ref, xy_ref, rx_ref, ry_ref):
    scale = jnp.float32(2.0 / 3.0)
    rx_ref[...] = xx_ref[...].astype(jnp.float32) * scale - 1.0
    ry_ref[...] = xy_ref[...].astype(jnp.float32) * scale - 1.0


def kernel(x_x, x_y):
    xt = x_x.T
    yt = x_y.T
    grid = (_H // _HBLK,)
    spec = pl.BlockSpec((_HBLK, _B), lambda i: (i, 0))
    rx, ry = pl.pallas_call(
        _body,
        grid=grid,
        in_specs=[spec, spec],
        out_specs=[spec, spec],
        out_shape=[jax.ShapeDtypeStruct((_H, _B), jnp.float32)] * 2,
    )(xt, yt)
    re = jnp.stack((rx, ry), axis=1)
    out3 = re.astype(jnp.complex64)
    return out3.transpose(2, 0, 1)


# speedup vs baseline: 68.8633x; 1.1773x over previous
import jax
import jax.numpy as jnp
from jax.experimental import pallas as pl

_B, _H = 16384, 200


def _body(xx_ref, xy_ref, o_ref):
    scale = jnp.float32(2.0 / 3.0)
    x3 = xx_ref[...].reshape(8, 128, 128)
    y3 = xy_ref[...].reshape(8, 128, 128)
    fx = x3.astype(jnp.float32) * scale - 1.0
    fy = y3.astype(jnp.float32) * scale - 1.0
    o_ref[...] = jnp.stack((fx, fy), axis=2).reshape(2048, 128)


def kernel(x_x, x_y):
    xt = x_x.T
    yt = x_y.T
    ispec = pl.BlockSpec((8, _B), lambda i: (i, 0))
    ospec = pl.BlockSpec((2048, 128), lambda i: (i, 0))
    f = pl.pallas_call(
        _body,
        grid=(_H // 8,),
        in_specs=[ispec, ispec],
        out_specs=ospec,
        out_shape=jax.ShapeDtypeStruct((_H * 256, 128), jnp.float32),
    )(xt, yt)
    cf = f.astype(jnp.complex64)
    out = cf.reshape(_H, 128, 2, 128).transpose(1, 3, 0, 2)
    return out.reshape(_B, _H, 2)
